# baseline (speedup 1.0000x reference)
import jax
import jax.numpy as jnp
from jax import lax
from jax.experimental import pallas as pl
from jax.experimental.pallas import tpu as pltpu
from jax.experimental.pallas import tpu_sc as plsc

_NUM_CORES = 2
_NUM_SUBCORES = 16
_NUM_WORKERS = _NUM_CORES * _NUM_SUBCORES


def _copy_body(emb_hbm, out_hbm):
    wid = lax.axis_index("s") * _NUM_CORES + lax.axis_index("c")
    rows = emb_hbm.shape[0] // _NUM_WORKERS
    base = wid * rows
    pltpu.sync_copy(emb_hbm.at[pl.ds(base, rows)], out_hbm.at[pl.ds(base, rows)])


def kernel(x, emb):
    seq_len = x.shape[1]
    mesh = plsc.VectorSubcoreMesh(core_axis_name="c", subcore_axis_name="s")
    out = pl.kernel(
        _copy_body,
        out_type=jax.ShapeDtypeStruct((seq_len, emb.shape[1]), emb.dtype),
        mesh=mesh,
    )(emb[:seq_len])
    return out[None]


# speedup vs baseline: 1.0045x; 1.0045x over previous
import jax
import jax.numpy as jnp
from jax import lax
from jax.experimental import pallas as pl
from jax.experimental.pallas import tpu as pltpu
from jax.experimental.pallas import tpu_sc as plsc

_NUM_CORES = 2
_NUM_SUBCORES = 16
_NUM_WORKERS = _NUM_CORES * _NUM_SUBCORES


_CHUNKS_PER_WORKER = 16


def _copy_body(emb_hbm, out_hbm, sem):
    wid = lax.axis_index("s") * _NUM_CORES + lax.axis_index("c")
    rows = emb_hbm.shape[0] // _NUM_WORKERS
    base = wid * rows
    chunk = rows // _CHUNKS_PER_WORKER
    copies = []
    for i in range(_CHUNKS_PER_WORKER):
        lo = base + i * chunk
        copies.append(
            pltpu.async_copy(
                emb_hbm.at[pl.ds(lo, chunk)], out_hbm.at[pl.ds(lo, chunk)], sem
            )
        )
    for c in copies:
        c.wait()


def kernel(x, emb):
    seq_len = x.shape[1]
    mesh = plsc.VectorSubcoreMesh(core_axis_name="c", subcore_axis_name="s")
    out = pl.kernel(
        _copy_body,
        out_type=jax.ShapeDtypeStruct((seq_len, emb.shape[1]), emb.dtype),
        mesh=mesh,
        scratch_types=[pltpu.SemaphoreType.DMA],
    )(emb[:seq_len])
    return out[None]


# speedup vs baseline: 30.2104x; 30.0748x over previous
import jax
import jax.numpy as jnp
from jax import lax
from jax.experimental import pallas as pl
from jax.experimental.pallas import tpu as pltpu
from jax.experimental.pallas import tpu_sc as plsc

_NUM_CORES = 2
_NUM_SUBCORES = 16
_NUM_WORKERS = _NUM_CORES * _NUM_SUBCORES


_CHUNK_ROWS = 16
_NBUF = 2


def _copy_body(emb_hbm, out_hbm, buf0, buf1, isem0, isem1, osem0, osem1):
    wid = lax.axis_index("s") * _NUM_CORES + lax.axis_index("c")
    rows = emb_hbm.shape[0] // _NUM_WORKERS
    base = wid * rows
    nchunks = rows // _CHUNK_ROWS
    bufs = [buf0, buf1]
    isems = [isem0, isem1]
    osems = [osem0, osem1]
    in_c = [None] * _NBUF
    out_c = [None] * _NBUF
    for i in range(nchunks):
        b = i % _NBUF
        if out_c[b] is not None:
            out_c[b].wait()
        lo = base + i * _CHUNK_ROWS
        in_c[b] = pltpu.async_copy(
            emb_hbm.at[pl.ds(lo, _CHUNK_ROWS)], bufs[b], isems[b]
        )
        in_c[b].wait()
        out_c[b] = pltpu.async_copy(
            bufs[b], out_hbm.at[pl.ds(lo, _CHUNK_ROWS)], osems[b]
        )
    for b in range(_NBUF):
        if out_c[b] is not None:
            out_c[b].wait()


def kernel(x, emb):
    seq_len = x.shape[1]
    d = emb.shape[1]
    mesh = plsc.VectorSubcoreMesh(core_axis_name="c", subcore_axis_name="s")
    out = pl.kernel(
        _copy_body,
        out_type=jax.ShapeDtypeStruct((seq_len, d), emb.dtype),
        mesh=mesh,
        scratch_types=[
            pltpu.VMEM((_CHUNK_ROWS, d), jnp.float32),
            pltpu.VMEM((_CHUNK_ROWS, d), jnp.float32),
            pltpu.SemaphoreType.DMA,
            pltpu.SemaphoreType.DMA,
            pltpu.SemaphoreType.DMA,
            pltpu.SemaphoreType.DMA,
        ],
    )(emb[:seq_len])
    return out[None]


# speedup vs baseline: 30.8781x; 1.0221x over previous
import jax
import jax.numpy as jnp
from jax import lax
from jax.experimental import pallas as pl
from jax.experimental.pallas import tpu as pltpu
from jax.experimental.pallas import tpu_sc as plsc

_NUM_CORES = 2
_NUM_SUBCORES = 16
_NUM_WORKERS = _NUM_CORES * _NUM_SUBCORES


_CHUNK_ROWS = 8
_NBUF = 4


def _copy_body(emb_hbm, out_hbm, *scratch):
    bufs = list(scratch[:_NBUF])
    isems = list(scratch[_NBUF : 2 * _NBUF])
    osems = list(scratch[2 * _NBUF : 3 * _NBUF])
    wid = lax.axis_index("s") * _NUM_CORES + lax.axis_index("c")
    rows = emb_hbm.shape[0] // _NUM_WORKERS
    base = wid * rows
    nchunks = rows // _CHUNK_ROWS
    in_c = [None] * _NBUF
    out_c = [None] * _NBUF
    for i in range(nchunks):
        b = i % _NBUF
        if out_c[b] is not None:
            out_c[b].wait()
        lo = base + i * _CHUNK_ROWS
        in_c[b] = pltpu.async_copy(
            emb_hbm.at[pl.ds(lo, _CHUNK_ROWS)], bufs[b], isems[b]
        )
        if i > 0:
            pb = (i - 1) % _NBUF
            in_c[pb].wait()
            plo = base + (i - 1) * _CHUNK_ROWS
            out_c[pb] = pltpu.async_copy(
                bufs[pb], out_hbm.at[pl.ds(plo, _CHUNK_ROWS)], osems[pb]
            )
    lb = (nchunks - 1) % _NBUF
    in_c[lb].wait()
    llo = base + (nchunks - 1) * _CHUNK_ROWS
    out_c[lb] = pltpu.async_copy(
        bufs[lb], out_hbm.at[pl.ds(llo, _CHUNK_ROWS)], osems[lb]
    )
    for b in range(_NBUF):
        if out_c[b] is not None:
            out_c[b].wait()


def kernel(x, emb):
    seq_len = x.shape[1]
    d = emb.shape[1]
    mesh = plsc.VectorSubcoreMesh(core_axis_name="c", subcore_axis_name="s")
    out = pl.kernel(
        _copy_body,
        out_type=jax.ShapeDtypeStruct((seq_len, d), emb.dtype),
        mesh=mesh,
        scratch_types=(
            [pltpu.VMEM((_CHUNK_ROWS, d), jnp.float32)] * _NBUF
            + [pltpu.SemaphoreType.DMA] * (2 * _NBUF)
        ),
    )(emb[:seq_len])
    return out[None]


# speedup vs baseline: 30.9376x; 1.0019x over previous
import jax
import jax.numpy as jnp
from jax import lax
from jax.experimental import pallas as pl
from jax.experimental.pallas import tpu as pltpu
from jax.experimental.pallas import tpu_sc as plsc

_NUM_CORES = 2
_NUM_SUBCORES = 16
_NUM_WORKERS = _NUM_CORES * _NUM_SUBCORES


_CHUNK_ROWS = 16
_NBUF = 3


def _copy_body(emb_hbm, out_hbm, *scratch):
    bufs = list(scratch[:_NBUF])
    isems = list(scratch[_NBUF : 2 * _NBUF])
    osems = list(scratch[2 * _NBUF : 3 * _NBUF])
    wid = lax.axis_index("s") * _NUM_CORES + lax.axis_index("c")
    rows = emb_hbm.shape[0] // _NUM_WORKERS
    base = wid * rows
    nchunks = rows // _CHUNK_ROWS
    in_c = [None] * _NBUF
    out_c = [None] * _NBUF
    for i in range(nchunks):
        b = i % _NBUF
        if out_c[b] is not None:
            out_c[b].wait()
        lo = base + i * _CHUNK_ROWS
        in_c[b] = pltpu.async_copy(
            emb_hbm.at[pl.ds(lo, _CHUNK_ROWS)], bufs[b], isems[b]
        )
        if i > 0:
            pb = (i - 1) % _NBUF
            in_c[pb].wait()
            plo = base + (i - 1) * _CHUNK_ROWS
            out_c[pb] = pltpu.async_copy(
                bufs[pb], out_hbm.at[pl.ds(plo, _CHUNK_ROWS)], osems[pb]
            )
    lb = (nchunks - 1) % _NBUF
    in_c[lb].wait()
    llo = base + (nchunks - 1) * _CHUNK_ROWS
    out_c[lb] = pltpu.async_copy(
        bufs[lb], out_hbm.at[pl.ds(llo, _CHUNK_ROWS)], osems[lb]
    )
    for b in range(_NBUF):
        if out_c[b] is not None:
            out_c[b].wait()


def kernel(x, emb):
    seq_len = x.shape[1]
    d = emb.shape[1]
    mesh = plsc.VectorSubcoreMesh(core_axis_name="c", subcore_axis_name="s")
    out = pl.kernel(
        _copy_body,
        out_type=jax.ShapeDtypeStruct((seq_len, d), emb.dtype),
        mesh=mesh,
        scratch_types=(
            [pltpu.VMEM((_CHUNK_ROWS, d), jnp.float32)] * _NBUF
            + [pltpu.SemaphoreType.DMA] * (2 * _NBUF)
        ),
    )(emb[:seq_len])
    return out[None]


# speedup vs baseline: 47.1517x; 1.5241x over previous
import jax
import jax.numpy as jnp
from jax.experimental import pallas as pl
from jax.experimental.pallas import tpu as pltpu

_BLOCK_ROWS = 512

def _copy_block(emb_ref, out_ref):
    out_ref[...] = emb_ref[...]

def kernel(x, emb):
    seq_len = x.shape[1]
    d = emb.shape[1]
    grid = seq_len // _BLOCK_ROWS
    out = pl.pallas_call(
        _copy_block,
        grid=(grid,),
        in_specs=[pl.BlockSpec((_BLOCK_ROWS, d), lambda i: (i, 0))],
        out_specs=pl.BlockSpec((_BLOCK_ROWS, d), lambda i: (i, 0)),
        out_shape=jax.ShapeDtypeStruct((seq_len, d), emb.dtype),
    )(emb[:seq_len])
    return out[None]
